# Initial kernel scaffold; baseline (speedup 1.0000x reference)
import jax
import jax.numpy as jnp
from jax.experimental import pallas as pl


def kernel(lb, ub, node_id):
    raise NotImplementedError("write your pallas kernel here")


# speedup vs baseline: 3.2426x; 3.2426x over previous
import functools

import jax
import jax.numpy as jnp
from jax import lax
from jax.experimental import pallas as pl
from jax.experimental.pallas import tpu as pltpu
from jax.experimental.pallas import tpu_sc as plsc

_NUM_CORES = 2
_NUM_SUBCORES = 16
_LANES = 16


@functools.cache
def _make_sc_kernel(N: int, D: int, C: int):
    NW = _NUM_CORES * _NUM_SUBCORES
    NCHUNKS = N // C
    TMAX = (NCHUNKS + NW - 1) // NW
    JV = D // _LANES

    mesh = plsc.VectorSubcoreMesh(
        core_axis_name="c", subcore_axis_name="s",
        num_cores=_NUM_CORES, num_subcores=_NUM_SUBCORES)

    def body(lb_hbm, ub_hbm, nid_hbm, out_hbm, idx_v, lb_v, ub_v, sem):
        wid = lax.axis_index("s") * _NUM_CORES + lax.axis_index("c")

        def do_chunk(g):
            base = g * C
            pltpu.sync_copy(nid_hbm.at[pl.ds(base, C)], idx_v)
            cp_l = pltpu.async_copy(lb_hbm.at[idx_v], lb_v, sem)
            cp_u = pltpu.async_copy(ub_hbm.at[idx_v], ub_v, sem)
            cp_l.wait()
            cp_u.wait()

            def row_body(r, carry):
                for j in range(JV):
                    sl = pl.ds(j * _LANES, _LANES)
                    l = lb_v[r, sl]
                    u = ub_v[r, sl]
                    zero = jnp.zeros_like(l)
                    pos = l > zero
                    sel = (l < zero) & (u > zero)
                    keep_u = pos | sel
                    keep_l = pos | (sel & ((l + u) > zero))
                    ub_v[r, sl] = jnp.where(keep_u, u, zero)
                    lb_v[r, sl] = jnp.where(keep_l, l, zero)
                return carry

            lax.fori_loop(0, C, row_body, 0)
            pltpu.sync_copy(lb_v, out_hbm.at[0, pl.ds(base, C)])
            pltpu.sync_copy(ub_v, out_hbm.at[1, pl.ds(base, C)])

        def step(t, carry):
            g = t * NW + wid

            @pl.when(g < NCHUNKS)
            def _():
                do_chunk(g)

            return carry

        lax.fori_loop(0, TMAX, step, 0)

    return pl.kernel(
        body,
        out_type=jax.ShapeDtypeStruct((2, N, D), jnp.float32),
        mesh=mesh,
        scratch_types=[
            pltpu.VMEM((C,), jnp.int32),
            pltpu.VMEM((C, D), jnp.float32),
            pltpu.VMEM((C, D), jnp.float32),
            pltpu.SemaphoreType.DMA,
        ],
    )


def kernel(lb, ub, node_id):
    N, D = lb.shape
    C = next(c for c in (40, 80, 16, 8) if N % c == 0 and c * D * 8 <= 400_000)
    return _make_sc_kernel(N, D, C)(lb, ub, node_id.astype(jnp.int32))


# speedup vs baseline: 4.0047x; 1.2350x over previous
import functools

import jax
import jax.numpy as jnp
from jax import lax
from jax.experimental import pallas as pl
from jax.experimental.pallas import tpu as pltpu
from jax.experimental.pallas import tpu_sc as plsc

_NUM_CORES = 2
_NUM_SUBCORES = 16
_LANES = 16
_NBUF = 3


@functools.cache
def _make_sc_kernel(N: int, D: int, C: int):
    NW = _NUM_CORES * _NUM_SUBCORES
    NCHUNKS = N // C
    TMAX = (NCHUNKS + NW - 1) // NW
    NTB = (TMAX + 1 + _NBUF - 1) // _NBUF
    JV = D // _LANES

    mesh = plsc.VectorSubcoreMesh(
        core_axis_name="c", subcore_axis_name="s",
        num_cores=_NUM_CORES, num_subcores=_NUM_SUBCORES)

    def body(lb_hbm, ub_hbm, nid_hbm, out_hbm, idx_v, lb_v, ub_v, gsem, ssem):
        wid = lax.axis_index("s") * _NUM_CORES + lax.axis_index("c")

        def in_range(t):
            return (t * NW + wid) < NCHUNKS

        def gather_descs(p):
            return (pltpu.make_async_copy(lb_hbm.at[idx_v.at[p]], lb_v.at[p],
                                          gsem.at[p]),
                    pltpu.make_async_copy(ub_hbm.at[idx_v.at[p]], ub_v.at[p],
                                          gsem.at[p]))

        def scatter_descs(g, p):
            base = g * C
            return (pltpu.make_async_copy(lb_v.at[p],
                                          out_hbm.at[0, pl.ds(base, C)],
                                          ssem.at[p]),
                    pltpu.make_async_copy(ub_v.at[p],
                                          out_hbm.at[1, pl.ds(base, C)],
                                          ssem.at[p]))

        def start_gather(t, p):
            @pl.when(in_range(t))
            def _():
                g = t * NW + wid
                pltpu.sync_copy(nid_hbm.at[pl.ds(g * C, C)], idx_v.at[p])
                for d in gather_descs(p):
                    d.start()

        def wait_gather(t, p):
            @pl.when(in_range(t))
            def _():
                for d in gather_descs(p):
                    d.wait()

        def start_scatter(t, p):
            @pl.when(in_range(t))
            def _():
                g = t * NW + wid
                for d in scatter_descs(g, p):
                    d.start()

        def wait_scatter(t, p):
            @pl.when((t >= 0) & in_range(t))
            def _():
                g = t * NW + wid
                for d in scatter_descs(g, p):
                    d.wait()

        def compute(t, p):
            @pl.when(in_range(t))
            def _():
                def row_body(r, carry):
                    for j in range(JV):
                        sl = pl.ds(j * _LANES, _LANES)
                        l = lb_v[p, r, sl]
                        u = ub_v[p, r, sl]
                        zero = jnp.zeros_like(l)
                        keep_u = (u > zero) & (l != zero)
                        keep_l = keep_u & ((l + u) > zero)
                        ub_v[p, r, sl] = jnp.where(keep_u, u, zero)
                        lb_v[p, r, sl] = jnp.where(keep_l, l, zero)
                    return carry

                lax.fori_loop(0, C, row_body, 0)

        start_gather(0, 0)
        start_gather(1, 1)

        def block(tb, carry):
            for p in range(_NBUF):
                t = tb * _NBUF + p
                wait_gather(t, p)
                compute(t, p)
                start_scatter(t, p)
                pn = (p + 2) % _NBUF
                wait_scatter(t - 1, pn)
                start_gather(t + 2, pn)
            return carry

        lax.fori_loop(0, NTB, block, 0)

    return pl.kernel(
        body,
        out_type=jax.ShapeDtypeStruct((2, N, D), jnp.float32),
        mesh=mesh,
        scratch_types=[
            pltpu.VMEM((_NBUF, C), jnp.int32),
            pltpu.VMEM((_NBUF, C, D), jnp.float32),
            pltpu.VMEM((_NBUF, C, D), jnp.float32),
            pltpu.SemaphoreType.DMA((_NBUF,)),
            pltpu.SemaphoreType.DMA((_NBUF,)),
        ],
    )


def kernel(lb, ub, node_id):
    N, D = lb.shape
    C = next(c for c in (40, 16, 8) if N % c == 0 and c * D * 4 * 2 * _NBUF <= 500_000)
    return _make_sc_kernel(N, D, C)(lb, ub, node_id.astype(jnp.int32))


# speedup vs baseline: 4.6541x; 1.1621x over previous
import functools

import jax
import jax.numpy as jnp
from jax import lax
from jax.experimental import pallas as pl
from jax.experimental.pallas import tpu as pltpu
from jax.experimental.pallas import tpu_sc as plsc

_NUM_CORES = 2
_NUM_SUBCORES = 16
_LANES = 16
_NBUF = 3


@functools.cache
def _make_sc_kernel(N: int, D: int, C: int):
    NW = _NUM_CORES * _NUM_SUBCORES
    NCHUNKS = N // C
    TMAX = (NCHUNKS + NW - 1) // NW
    NTB = (TMAX + 1 + _NBUF - 1) // _NBUF
    JV = D // _LANES

    mesh = plsc.VectorSubcoreMesh(
        core_axis_name="c", subcore_axis_name="s",
        num_cores=_NUM_CORES, num_subcores=_NUM_SUBCORES)

    def body(lb_hbm, ub_hbm, nid_hbm, out_hbm, idx_v, lb_v, ub_v,
             gsem, ssem, isem):
        wid = lax.axis_index("s") * _NUM_CORES + lax.axis_index("c")

        def in_range(t):
            return (t * NW + wid) < NCHUNKS

        def idx_desc(t, p):
            g = t * NW + wid
            return pltpu.make_async_copy(nid_hbm.at[pl.ds(g * C, C)],
                                         idx_v.at[p], isem.at[p])

        def start_idx(t, p):
            @pl.when(in_range(t))
            def _():
                idx_desc(t, p).start()

        def wait_idx(t, p):
            @pl.when(in_range(t))
            def _():
                idx_desc(t, p).wait()

        def gather_descs(p):
            return (pltpu.make_async_copy(lb_hbm.at[idx_v.at[p]], lb_v.at[p],
                                          gsem.at[p]),
                    pltpu.make_async_copy(ub_hbm.at[idx_v.at[p]], ub_v.at[p],
                                          gsem.at[p]))

        def scatter_descs(g, p):
            base = g * C
            return (pltpu.make_async_copy(lb_v.at[p],
                                          out_hbm.at[0, pl.ds(base, C)],
                                          ssem.at[p]),
                    pltpu.make_async_copy(ub_v.at[p],
                                          out_hbm.at[1, pl.ds(base, C)],
                                          ssem.at[p]))

        def start_gather(t, p):
            @pl.when(in_range(t))
            def _():
                for d in gather_descs(p):
                    d.start()

        def wait_gather(t, p):
            @pl.when(in_range(t))
            def _():
                for d in gather_descs(p):
                    d.wait()

        def start_scatter(t, p):
            @pl.when(in_range(t))
            def _():
                g = t * NW + wid
                for d in scatter_descs(g, p):
                    d.start()

        def wait_scatter(t, p):
            @pl.when((t >= 0) & in_range(t))
            def _():
                g = t * NW + wid
                for d in scatter_descs(g, p):
                    d.wait()

        def compute(t, p):
            @pl.when(in_range(t))
            def _():
                def row_body(r, carry):
                    for j in range(JV):
                        sl = pl.ds(j * _LANES, _LANES)
                        l = lb_v[p, r, sl]
                        u = ub_v[p, r, sl]
                        zero = jnp.zeros_like(l)
                        ub_v[p, r, sl] = jnp.where(l == zero, zero,
                                                   jnp.maximum(u, zero))
                        lb_v[p, r, sl] = jnp.where((l + u) > zero, l, zero)
                    return carry

                lax.fori_loop(0, C, row_body, 0)

        start_idx(0, 0)
        start_idx(1, 1)
        start_idx(2, 2)
        wait_idx(0, 0)
        start_gather(0, 0)
        wait_idx(1, 1)
        start_gather(1, 1)

        def block(tb, carry):
            for p in range(_NBUF):
                t = tb * _NBUF + p
                wait_gather(t, p)
                compute(t, p)
                start_scatter(t, p)
                pn = (p + 2) % _NBUF
                wait_scatter(t - 1, pn)
                wait_idx(t + 2, pn)
                start_gather(t + 2, pn)
                start_idx(t + 3, p)
            return carry

        lax.fori_loop(0, NTB, block, 0)

    return pl.kernel(
        body,
        out_type=jax.ShapeDtypeStruct((2, N, D), jnp.float32),
        mesh=mesh,
        scratch_types=[
            pltpu.VMEM((_NBUF, C), jnp.int32),
            pltpu.VMEM((_NBUF, C, D), jnp.float32),
            pltpu.VMEM((_NBUF, C, D), jnp.float32),
            pltpu.SemaphoreType.DMA((_NBUF,)),
            pltpu.SemaphoreType.DMA((_NBUF,)),
            pltpu.SemaphoreType.DMA((_NBUF,)),
        ],
    )


def kernel(lb, ub, node_id):
    N, D = lb.shape
    C = next(c for c in (40, 16, 8) if N % c == 0 and c * D * 4 * 2 * _NBUF <= 500_000)
    return _make_sc_kernel(N, D, C)(lb, ub, node_id.astype(jnp.int32))
